# Initial kernel scaffold; baseline (speedup 1.0000x reference)
import jax
import jax.numpy as jnp
from jax.experimental import pallas as pl


def kernel(pos, batch, nn1_w1, nn1_b1, nn1_w2, nn1_b2, root1, bias1, nn2_w1, nn2_b1, nn2_w2, nn2_b2, root2, bias2, nn3_w1, nn3_b1, nn3_w2, nn3_b2, root3, bias3, lin1_w, lin1_b, lin2_w, lin2_b, lin3_w, lin3_b):
    raise NotImplementedError("write your pallas kernel here")


# speedup vs baseline: 1.0617x; 1.0617x over previous
import math

import jax
import jax.numpy as jnp
import numpy as np
from jax.experimental import pallas as pl

_HI = jax.lax.Precision.HIGHEST


def _ceil_to(x, m):
    return ((x + m - 1) // m) * m


def _d2_body(pos_blk, pos_all, sq_blk, sq_all, b_blk, b_all, v_blk, v_all, o_ref):
    blk, npad = o_ref.shape
    pr = pos_blk[...]
    pc = pos_all[...]
    d2 = sq_blk[...] + sq_all[...] - 2.0 * jax.lax.dot_general(
        pr, pc, (((1,), (1,)), ((), ())), precision=_HI,
        preferred_element_type=jnp.float32)
    i = pl.program_id(0)
    rows = i * blk + jax.lax.broadcasted_iota(jnp.int32, (blk, npad), 0)
    cols = jax.lax.broadcasted_iota(jnp.int32, (blk, npad), 1)
    mask = (b_blk[...] == b_all[...]) & (v_blk[...] > 0) & (v_all[...] > 0) \
        & (rows != cols)
    o_ref[...] = jnp.where(mask, d2, jnp.inf)


def _pairwise_d2(pos, batch, valid):
    n = pos.shape[0]
    blk = 256
    npad = _ceil_to(n, blk)
    pad = npad - n
    pos_p = jnp.pad(pos, ((0, pad), (0, 0)))
    sq = (pos_p * pos_p).sum(axis=1)
    b_p = jnp.pad(batch, (0, pad), constant_values=-1)
    v_p = jnp.pad(valid.astype(jnp.int32), (0, pad))
    grid = npad // blk
    d2 = pl.pallas_call(
        _d2_body,
        grid=(grid,),
        in_specs=[
            pl.BlockSpec((blk, 3), lambda i: (i, 0)),
            pl.BlockSpec((npad, 3), lambda i: (0, 0)),
            pl.BlockSpec((blk, 1), lambda i: (i, 0)),
            pl.BlockSpec((1, npad), lambda i: (0, 0)),
            pl.BlockSpec((blk, 1), lambda i: (i, 0)),
            pl.BlockSpec((1, npad), lambda i: (0, 0)),
            pl.BlockSpec((blk, 1), lambda i: (i, 0)),
            pl.BlockSpec((1, npad), lambda i: (0, 0)),
        ],
        out_specs=pl.BlockSpec((blk, npad), lambda i: (i, 0)),
        out_shape=jax.ShapeDtypeStruct((npad, npad), jnp.float32),
    )(pos_p, pos_p, sq.reshape(npad, 1), sq.reshape(1, npad),
      b_p.reshape(npad, 1), b_p.reshape(1, npad),
      v_p.reshape(npad, 1), v_p.reshape(1, npad))
    return d2[:n, :n]


def _radius_graph(pos, r, batch, valid, max_num_neighbors=32):
    n = pos.shape[0]
    k = min(max_num_neighbors, n - 1)
    d2 = _pairwise_d2(pos, batch, valid)
    rows = jnp.arange(n)
    negd, nbr = jax.lax.top_k(-d2, k)
    nd2 = -negd
    t = np.float64(r) * np.float64(r)
    t32 = np.float32(t)
    if t32 > t:
        t32 = np.nextafter(t32, np.float32(0.0))
    edge_ok = nd2 <= t32
    src = nbr.reshape(-1)
    tgt = jnp.repeat(rows, k)
    return src, tgt, edge_ok.reshape(-1)


def _make_msg_body(in_ch, out_ch):
    def body(ps_ref, xs_ref, em_ref, w1_ref, b1_ref, w2_ref, b2_ref, o_ref):
        z = jnp.maximum(
            jax.lax.dot_general(ps_ref[...], w1_ref[...],
                                (((1,), (0,)), ((), ())), precision=_HI,
                                preferred_element_type=jnp.float32)
            + b1_ref[...], 0.0)
        xs = xs_ref[...]
        acc = jax.lax.dot_general(xs, b2_ref[...], (((1,), (0,)), ((), ())),
                                  precision=_HI,
                                  preferred_element_type=jnp.float32)
        for i in range(in_ch):
            wi = w2_ref[:, i * out_ch:(i + 1) * out_ch]
            acc = acc + xs[:, i:i + 1] * jax.lax.dot_general(
                z, wi, (((1,), (0,)), ((), ())), precision=_HI,
                preferred_element_type=jnp.float32)
        o_ref[...] = acc * em_ref[...]
    return body


def _edge_messages(pseudo, xs, emf, w1, b1, w2, b2, in_ch, out_ch):
    e = pseudo.shape[0]
    eb = 512
    ep = _ceil_to(e, eb)
    pad = ep - e
    ps_p = jnp.pad(pseudo, ((0, pad), (0, 0)))
    xs_p = jnp.pad(xs, ((0, pad), (0, 0)))
    em_p = jnp.pad(emf.reshape(e, 1), ((0, pad), (0, 0)))
    hid = w1.shape[1]
    msg = pl.pallas_call(
        _make_msg_body(in_ch, out_ch),
        grid=(ep // eb,),
        in_specs=[
            pl.BlockSpec((eb, 3), lambda i: (i, 0)),
            pl.BlockSpec((eb, in_ch), lambda i: (i, 0)),
            pl.BlockSpec((eb, 1), lambda i: (i, 0)),
            pl.BlockSpec((3, hid), lambda i: (0, 0)),
            pl.BlockSpec((1, hid), lambda i: (0, 0)),
            pl.BlockSpec((hid, in_ch * out_ch), lambda i: (0, 0)),
            pl.BlockSpec((in_ch, out_ch), lambda i: (0, 0)),
        ],
        out_specs=pl.BlockSpec((eb, out_ch), lambda i: (i, 0)),
        out_shape=jax.ShapeDtypeStruct((ep, out_ch), jnp.float32),
    )(ps_p, xs_p, em_p, w1, b1.reshape(1, hid), w2,
      b2.reshape(in_ch, out_ch))
    return msg[:e]


def _nn_conv(x, pos, src, tgt, em, w1, b1, w2, b2, root, bias, in_ch, out_ch):
    pseudo = pos[tgt] - pos[src]
    xs = x[src]
    emf = em.astype(jnp.float32)
    msg = _edge_messages(pseudo, xs, emf, w1, b1, w2, b2, in_ch, out_ch)
    n = x.shape[0]
    s = jnp.zeros((n, out_ch), x.dtype).at[tgt].add(msg)
    c = jnp.zeros((n,), x.dtype).at[tgt].add(emf)
    agg = s / jnp.maximum(c, 1.0)[:, None]
    return jax.nn.relu(agg + x @ root + bias)


def _fps_sample(pos, batch, valid, ratio, num_batches):
    n = pos.shape[0]
    maxm = int(math.ceil(ratio * n))
    bids = jnp.arange(num_batches)
    mask = valid[None, :] & (batch[None, :] == bids[:, None])
    cnt = mask.sum(axis=1)
    m = jnp.where(cnt > 0,
                  jnp.maximum(1, jnp.ceil(ratio * cnt.astype(jnp.float32)).astype(jnp.int32)),
                  0)
    j0 = jnp.argmax(mask, axis=1)
    d0 = ((pos[None, :, :] - pos[j0][:, None, :]) ** 2).sum(axis=-1)
    dist = jnp.where(mask, d0, -jnp.inf)
    sel0 = jnp.zeros((num_batches, maxm), jnp.int32).at[:, 0].set(j0.astype(jnp.int32))

    def body(i, carry):
        dist, sel = carry
        j = jnp.argmax(dist, axis=1)
        sel = sel.at[:, i].set(j.astype(jnp.int32))
        nd = ((pos[None, :, :] - pos[j][:, None, :]) ** 2).sum(axis=-1)
        return jnp.minimum(dist, nd), sel

    _, sel = jax.lax.fori_loop(1, maxm, body, (dist, sel0))
    off = jnp.concatenate([jnp.zeros((1,), m.dtype), jnp.cumsum(m)[:-1]])
    cols = jnp.arange(maxm)
    dest = jnp.where(cols[None, :] < m[:, None], off[:, None] + cols[None, :], n)
    idx = jnp.zeros((n,), jnp.int32).at[dest.reshape(-1)].set(sel.reshape(-1), mode='drop')
    new_valid = jnp.arange(n) < m.sum()
    return idx, new_valid


def kernel(pos, batch, nn1_w1, nn1_b1, nn1_w2, nn1_b2, root1, bias1,
           nn2_w1, nn2_b1, nn2_w2, nn2_b2, root2, bias2,
           nn3_w1, nn3_b1, nn3_w2, nn3_b2, root3, bias3,
           lin1_w, lin1_b, lin2_w, lin2_b, lin3_w, lin3_b):
    n0 = pos.shape[0]
    nb = 8
    p1 = (n0 + nb) // 2
    p2 = (p1 + 3 * nb) // 4
    batch = batch.astype(jnp.int32)
    x = jnp.ones((n0, 1), jnp.float32)
    valid = jnp.ones((n0,), bool)

    src, tgt, em = _radius_graph(pos, 0.2, batch, valid)
    x = _nn_conv(x, pos, src, tgt, em, nn1_w1, nn1_b1, nn1_w2, nn1_b2,
                 root1, bias1, 1, 64)
    idx, valid = _fps_sample(pos, batch, valid, 0.5, nb)
    idx, valid = idx[:p1], valid[:p1]
    x, pos, batch = x[idx], pos[idx], batch[idx]

    src, tgt, em = _radius_graph(pos, 0.4, batch, valid)
    x = _nn_conv(x, pos, src, tgt, em, nn2_w1, nn2_b1, nn2_w2, nn2_b2,
                 root2, bias2, 64, 64)
    idx, valid = _fps_sample(pos, batch, valid, 0.25, nb)
    idx, valid = idx[:p2], valid[:p2]
    x, pos, batch = x[idx], pos[idx], batch[idx]

    src, tgt, em = _radius_graph(pos, 1.0, batch, valid)
    x = _nn_conv(x, pos, src, tgt, em, nn3_w1, nn3_b1, nn3_w2, nn3_b2,
                 root3, bias3, 64, 128)

    ng = nb
    s = jnp.zeros((ng, 128), x.dtype).at[batch].add(jnp.where(valid[:, None], x, 0.0))
    c = jnp.zeros((ng,), x.dtype).at[batch].add(jnp.where(valid, 1.0, 0.0))
    g = s / jnp.maximum(c, 1.0)[:, None]
    h = jax.nn.relu(g @ lin1_w + lin1_b)
    h = jax.nn.relu(h @ lin2_w + lin2_b)
    h = h @ lin3_w + lin3_b
    return jax.nn.log_softmax(h, axis=-1)


# speedup vs baseline: 1.5172x; 1.4290x over previous
import math

import jax
import jax.numpy as jnp
import numpy as np
from jax.experimental import pallas as pl
from jax.experimental.pallas import tpu as pltpu

_HI = jax.lax.Precision.HIGHEST


def _ceil_to(x, m):
    return ((x + m - 1) // m) * m


def _d2_body(pos_blk, pos_all, sq_blk, sq_all, b_blk, b_all, v_blk, v_all, o_ref):
    blk, npad = o_ref.shape
    pr = pos_blk[...]
    pc = pos_all[...]
    d2 = sq_blk[...] + sq_all[...] - 2.0 * jax.lax.dot_general(
        pr, pc, (((1,), (1,)), ((), ())), precision=_HI,
        preferred_element_type=jnp.float32)
    i = pl.program_id(0)
    rows = i * blk + jax.lax.broadcasted_iota(jnp.int32, (blk, npad), 0)
    cols = jax.lax.broadcasted_iota(jnp.int32, (blk, npad), 1)
    mask = (b_blk[...] == b_all[...]) & (v_blk[...] > 0) & (v_all[...] > 0) \
        & (rows != cols)
    o_ref[...] = jnp.where(mask, d2, jnp.inf)


def _pairwise_d2(pos, batch, valid):
    n = pos.shape[0]
    blk = 256
    npad = _ceil_to(n, blk)
    pad = npad - n
    pos_p = jnp.pad(pos, ((0, pad), (0, 0)))
    sq = (pos_p * pos_p).sum(axis=1)
    b_p = jnp.pad(batch, (0, pad), constant_values=-1)
    v_p = jnp.pad(valid.astype(jnp.int32), (0, pad))
    grid = npad // blk
    d2 = pl.pallas_call(
        _d2_body,
        grid=(grid,),
        in_specs=[
            pl.BlockSpec((blk, 3), lambda i: (i, 0)),
            pl.BlockSpec((npad, 3), lambda i: (0, 0)),
            pl.BlockSpec((blk, 1), lambda i: (i, 0)),
            pl.BlockSpec((1, npad), lambda i: (0, 0)),
            pl.BlockSpec((blk, 1), lambda i: (i, 0)),
            pl.BlockSpec((1, npad), lambda i: (0, 0)),
            pl.BlockSpec((blk, 1), lambda i: (i, 0)),
            pl.BlockSpec((1, npad), lambda i: (0, 0)),
        ],
        out_specs=pl.BlockSpec((blk, npad), lambda i: (i, 0)),
        out_shape=jax.ShapeDtypeStruct((npad, npad), jnp.float32),
    )(pos_p, pos_p, sq.reshape(npad, 1), sq.reshape(1, npad),
      b_p.reshape(npad, 1), b_p.reshape(1, npad),
      v_p.reshape(npad, 1), v_p.reshape(1, npad))
    return d2[:n, :n]


def _radius_graph(pos, r, batch, valid, max_num_neighbors=32):
    n = pos.shape[0]
    k = min(max_num_neighbors, n - 1)
    d2 = _pairwise_d2(pos, batch, valid)
    rows = jnp.arange(n)
    negd, nbr = jax.lax.top_k(-d2, k)
    nd2 = -negd
    t = np.float64(r) * np.float64(r)
    t32 = np.float32(t)
    if t32 > t:
        t32 = np.nextafter(t32, np.float32(0.0))
    edge_ok = nd2 <= t32
    src = nbr.reshape(-1)
    tgt = jnp.repeat(rows, k)
    return src, tgt, edge_ok.reshape(-1)


def _make_msg_body(in_ch, out_ch):
    def body(ps_ref, xs_ref, em_ref, w1_ref, b1_ref, w2_ref, b2_ref, o_ref):
        z = jnp.maximum(
            jax.lax.dot_general(ps_ref[...], w1_ref[...],
                                (((1,), (0,)), ((), ())), precision=_HI,
                                preferred_element_type=jnp.float32)
            + b1_ref[...], 0.0)
        xs = xs_ref[...]
        acc = jax.lax.dot_general(xs, b2_ref[...], (((1,), (0,)), ((), ())),
                                  precision=_HI,
                                  preferred_element_type=jnp.float32)
        for i in range(in_ch):
            wi = w2_ref[:, i * out_ch:(i + 1) * out_ch]
            acc = acc + xs[:, i:i + 1] * jax.lax.dot_general(
                z, wi, (((1,), (0,)), ((), ())), precision=_HI,
                preferred_element_type=jnp.float32)
        o_ref[...] = acc * em_ref[...]
    return body


def _edge_messages(pseudo, xs, emf, w1, b1, w2, b2, in_ch, out_ch):
    e = pseudo.shape[0]
    eb = 512
    ep = _ceil_to(e, eb)
    pad = ep - e
    ps_p = jnp.pad(pseudo, ((0, pad), (0, 0)))
    xs_p = jnp.pad(xs, ((0, pad), (0, 0)))
    em_p = jnp.pad(emf.reshape(e, 1), ((0, pad), (0, 0)))
    hid = w1.shape[1]
    msg = pl.pallas_call(
        _make_msg_body(in_ch, out_ch),
        grid=(ep // eb,),
        in_specs=[
            pl.BlockSpec((eb, 3), lambda i: (i, 0)),
            pl.BlockSpec((eb, in_ch), lambda i: (i, 0)),
            pl.BlockSpec((eb, 1), lambda i: (i, 0)),
            pl.BlockSpec((3, hid), lambda i: (0, 0)),
            pl.BlockSpec((1, hid), lambda i: (0, 0)),
            pl.BlockSpec((hid, in_ch * out_ch), lambda i: (0, 0)),
            pl.BlockSpec((in_ch, out_ch), lambda i: (0, 0)),
        ],
        out_specs=pl.BlockSpec((eb, out_ch), lambda i: (i, 0)),
        out_shape=jax.ShapeDtypeStruct((ep, out_ch), jnp.float32),
    )(ps_p, xs_p, em_p, w1, b1.reshape(1, hid), w2,
      b2.reshape(in_ch, out_ch))
    return msg[:e]


def _nn_conv(x, pos, src, tgt, em, w1, b1, w2, b2, root, bias, in_ch, out_ch):
    pseudo = pos[tgt] - pos[src]
    xs = x[src]
    emf = em.astype(jnp.float32)
    msg = _edge_messages(pseudo, xs, emf, w1, b1, w2, b2, in_ch, out_ch)
    n = x.shape[0]
    s = jnp.zeros((n, out_ch), x.dtype).at[tgt].add(msg)
    c = jnp.zeros((n,), x.dtype).at[tgt].add(emf)
    agg = s / jnp.maximum(c, 1.0)[:, None]
    return jax.nn.relu(agg + x @ root + bias)


def _make_fps_body(nb, npad, maxm):
    def body(pos_ref, post_ref, dist0_ref, j0_ref, sel_ref, dist_scr):
        dist_scr[...] = dist0_ref[...]
        sel_ref[pl.dslice(0, 1), :] = j0_ref[...]
        post = post_ref[...]

        def it(i, carry):
            dist = dist_scr[...]
            j = jnp.argmax(dist, axis=1).astype(jnp.int32)
            sel_ref[pl.dslice(i, 1), :] = j[None, :]
            oh = (jax.lax.broadcasted_iota(jnp.int32, (nb, npad), 1)
                  == j[:, None]).astype(jnp.float32)
            pj = jax.lax.dot_general(oh, pos_ref[...],
                                     (((1,), (0,)), ((), ())), precision=_HI,
                                     preferred_element_type=jnp.float32)
            nd = ((post[0:1, :] - pj[:, 0:1]) ** 2
                  + (post[1:2, :] - pj[:, 1:2]) ** 2
                  + (post[2:3, :] - pj[:, 2:3]) ** 2)
            dist_scr[...] = jnp.minimum(dist, nd)
            return carry

        jax.lax.fori_loop(1, maxm, it, 0)
    return body


def _fps_sample(pos, batch, valid, ratio, num_batches):
    n = pos.shape[0]
    maxm = int(math.ceil(ratio * n))
    bids = jnp.arange(num_batches)
    mask = valid[None, :] & (batch[None, :] == bids[:, None])
    cnt = mask.sum(axis=1)
    m = jnp.where(cnt > 0,
                  jnp.maximum(1, jnp.ceil(ratio * cnt.astype(jnp.float32)).astype(jnp.int32)),
                  0)
    j0 = jnp.argmax(mask, axis=1)
    d0 = ((pos[None, :, :] - pos[j0][:, None, :]) ** 2).sum(axis=-1)
    dist = jnp.where(mask, d0, -jnp.inf)

    nb = num_batches
    npad = _ceil_to(n, 128)
    pos_p = jnp.pad(pos, ((0, npad - n), (0, 0)))
    dist_p = jnp.pad(dist, ((0, 0), (0, npad - n)),
                     constant_values=-jnp.inf)
    selT = pl.pallas_call(
        _make_fps_body(nb, npad, maxm),
        out_shape=jax.ShapeDtypeStruct((maxm, nb), jnp.int32),
        scratch_shapes=[pltpu.VMEM((nb, npad), jnp.float32)],
    )(pos_p, pos_p.T, dist_p, j0.astype(jnp.int32).reshape(1, nb))
    sel = selT.T
    off = jnp.concatenate([jnp.zeros((1,), m.dtype), jnp.cumsum(m)[:-1]])
    cols = jnp.arange(maxm)
    dest = jnp.where(cols[None, :] < m[:, None], off[:, None] + cols[None, :], n)
    idx = jnp.zeros((n,), jnp.int32).at[dest.reshape(-1)].set(sel.reshape(-1), mode='drop')
    new_valid = jnp.arange(n) < m.sum()
    return idx, new_valid


def kernel(pos, batch, nn1_w1, nn1_b1, nn1_w2, nn1_b2, root1, bias1,
           nn2_w1, nn2_b1, nn2_w2, nn2_b2, root2, bias2,
           nn3_w1, nn3_b1, nn3_w2, nn3_b2, root3, bias3,
           lin1_w, lin1_b, lin2_w, lin2_b, lin3_w, lin3_b):
    n0 = pos.shape[0]
    nb = 8
    p1 = (n0 + nb) // 2
    p2 = (p1 + 3 * nb) // 4
    batch = batch.astype(jnp.int32)
    x = jnp.ones((n0, 1), jnp.float32)
    valid = jnp.ones((n0,), bool)

    src, tgt, em = _radius_graph(pos, 0.2, batch, valid)
    x = _nn_conv(x, pos, src, tgt, em, nn1_w1, nn1_b1, nn1_w2, nn1_b2,
                 root1, bias1, 1, 64)
    idx, valid = _fps_sample(pos, batch, valid, 0.5, nb)
    idx, valid = idx[:p1], valid[:p1]
    x, pos, batch = x[idx], pos[idx], batch[idx]

    src, tgt, em = _radius_graph(pos, 0.4, batch, valid)
    x = _nn_conv(x, pos, src, tgt, em, nn2_w1, nn2_b1, nn2_w2, nn2_b2,
                 root2, bias2, 64, 64)
    idx, valid = _fps_sample(pos, batch, valid, 0.25, nb)
    idx, valid = idx[:p2], valid[:p2]
    x, pos, batch = x[idx], pos[idx], batch[idx]

    src, tgt, em = _radius_graph(pos, 1.0, batch, valid)
    x = _nn_conv(x, pos, src, tgt, em, nn3_w1, nn3_b1, nn3_w2, nn3_b2,
                 root3, bias3, 64, 128)

    ng = nb
    s = jnp.zeros((ng, 128), x.dtype).at[batch].add(jnp.where(valid[:, None], x, 0.0))
    c = jnp.zeros((ng,), x.dtype).at[batch].add(jnp.where(valid, 1.0, 0.0))
    g = s / jnp.maximum(c, 1.0)[:, None]
    h = jax.nn.relu(g @ lin1_w + lin1_b)
    h = jax.nn.relu(h @ lin2_w + lin2_b)
    h = h @ lin3_w + lin3_b
    return jax.nn.log_softmax(h, axis=-1)
